# baseline (speedup 1.0000x reference)
import functools
import math

import jax
import jax.numpy as jnp
from jax import lax
from jax.experimental import pallas as pl
from jax.experimental.pallas import tpu as pltpu
from jax.experimental.pallas import tpu_sc as plsc

_RW = 192
_CH = 64
_NW = 32
_MESH = dict(core_axis_name="c", subcore_axis_name="s")


def _seg_attn_sc(qt, kt, vt, si, sj, bounds, nslots):
    a = qt.shape[1]
    lp = si.shape[0]
    b8 = bounds.shape[0]
    rows_out = nslots * _RW

    @functools.partial(
        pl.kernel,
        out_type=[jax.ShapeDtypeStruct((rows_out, a), jnp.float32),
                  jax.ShapeDtypeStruct((rows_out * 16,), jnp.float32)],
        mesh=plsc.VectorSubcoreMesh(**_MESH),
        compiler_params=pltpu.CompilerParams(needs_layout_passes=False),
        scratch_types=[
            pltpu.VMEM_SHARED((lp,), jnp.int32),
            pltpu.VMEM_SHARED((b8,), jnp.int32),
            pltpu.VMEM((_RW + 8, a), jnp.float32),
            pltpu.VMEM(((_RW + 8) * 16,), jnp.float32),
            pltpu.VMEM((_CH, a), jnp.float32),
            pltpu.VMEM((_CH, a), jnp.float32),
            pltpu.VMEM((_CH, a), jnp.float32),
            pltpu.VMEM((_CH,), jnp.int32),
            pltpu.VMEM((_CH,), jnp.int32),
            pltpu.VMEM((16,), jnp.float32),
            pltpu.SMEM((b8,), jnp.int32),
            pltpu.SMEM((_CH,), jnp.int32),
            pltpu.SemaphoreType.DMA,
            pltpu.SemaphoreType.DMA,
            pltpu.SemaphoreType.DMA,
        ],
    )
    def body(qt_h, kt_h, vt_h, si_h, sj_h, bnd_h, num_h, den_h,
             si_sh, bnd_sh, acc_num, acc_den, qrows, krows, vrows,
             si_v, sj_v, exv, bnd_s, si_s, sem_q, sem_k, sem_v):
        c = lax.axis_index("c")
        t = lax.axis_index("s")
        w = t * 2 + c
        lane = lax.iota(jnp.int32, 16)

        @pl.when(t == 0)
        def _():
            pltpu.sync_copy(si_h, si_sh)
            pltpu.sync_copy(bnd_h, bnd_sh)

        plsc.subcore_barrier()
        pltpu.sync_copy(bnd_sh, bnd_s)

        def slot(iu, _):
            u = w + _NW * iu
            base = u * _RW
            b0 = bnd_s[u]
            b1 = bnd_s[u + 1]

            def zrow(r, _):
                z = jnp.zeros((16,), jnp.float32)
                for cq in range(a // 16):
                    acc_num[r, pl.ds(16 * cq, 16)] = z
                acc_den[pl.ds(16 * r, 16)] = z
                return 0

            lax.fori_loop(0, _RW + 8, zrow, 0)

            a0 = (b0 // _CH) * _CH
            nch = (b1 - a0 + _CH - 1) // _CH

            def chunk(ic, _):
                off = a0 + ic * _CH
                pltpu.sync_copy(si_sh.at[pl.ds(off, _CH)], si_v)
                pltpu.sync_copy(sj_h.at[pl.ds(off, _CH)], sj_v)
                pltpu.sync_copy(si_sh.at[pl.ds(off, _CH)], si_s)
                cq = pltpu.async_copy(qt_h.at[si_v], qrows, sem_q)
                ck = pltpu.async_copy(kt_h.at[sj_v], krows, sem_k)
                cv = pltpu.async_copy(vt_h.at[sj_v], vrows, sem_v)
                cq.wait()
                ck.wait()
                cv.wait()

                def line(l, _):
                    li = off + l
                    ok = (li >= b0) & (li < b1)
                    loc = jnp.where(ok, si_s[l] - base, _RW)
                    acc = jnp.zeros((16,), jnp.float32)
                    for r in range(8):
                        m = (qrows[l, pl.ds(32 * r, 16)]
                             * krows[l, pl.ds(32 * r, 16)]
                             + qrows[l, pl.ds(32 * r + 16, 16)]
                             * krows[l, pl.ds(32 * r + 16, 16)])
                        acc = jnp.where(lane == r, jnp.sum(m), acc)
                    ex = jnp.exp(acc)
                    acc_den[pl.ds(loc * 16, 16)] = (
                        acc_den[pl.ds(loc * 16, 16)] + ex)
                    exv[pl.ds(0, 16)] = ex
                    for r in range(8):
                        er = plsc.load_gather(
                            exv, [jnp.full((16,), r, jnp.int32)])
                        for hh in range(2):
                            co = 32 * r + 16 * hh
                            acc_num[loc, pl.ds(co, 16)] = (
                                acc_num[loc, pl.ds(co, 16)]
                                + vrows[l, pl.ds(co, 16)] * er)
                    return 0

                lax.fori_loop(0, _CH, line, 0)
                return 0

            lax.fori_loop(0, nch, chunk, 0)
            pltpu.sync_copy(acc_num.at[pl.ds(0, _RW)],
                            num_h.at[pl.ds(base, _RW)])
            pltpu.sync_copy(acc_den.at[pl.ds(0, _RW * 16)],
                            den_h.at[pl.ds(base * 16, _RW * 16)])
            return 0

        lax.fori_loop(0, (nslots - w + _NW - 1) // _NW, slot, 0)

    return body(qt, kt, vt, si, sj, bounds)


def _pair_gather_sc(xx, i_idx, j_idx):
    e = i_idx.shape[0]
    a = xx.shape[1]
    nch = e // _CH

    @functools.partial(
        pl.kernel,
        out_type=[jax.ShapeDtypeStruct((e, a), jnp.float32),
                  jax.ShapeDtypeStruct((e, a), jnp.float32)],
        mesh=plsc.VectorSubcoreMesh(**_MESH),
        compiler_params=pltpu.CompilerParams(needs_layout_passes=False),
        scratch_types=[
            pltpu.VMEM((_CH,), jnp.int32),
            pltpu.VMEM((_CH,), jnp.int32),
            pltpu.VMEM((_CH, a), jnp.float32),
            pltpu.VMEM((_CH, a), jnp.float32),
            pltpu.SemaphoreType.DMA,
            pltpu.SemaphoreType.DMA,
        ],
    )
    def body(xx_h, ii_h, jj_h, xi_h, xj_h, iv, jv, ri, rj, si, sj):
        c = lax.axis_index("c")
        t = lax.axis_index("s")
        w = t * 2 + c
        cnt = (nch - w + _NW - 1) // _NW

        def chunk(i, _):
            off = (w + i * _NW) * _CH
            pltpu.sync_copy(ii_h.at[pl.ds(off, _CH)], iv)
            pltpu.sync_copy(jj_h.at[pl.ds(off, _CH)], jv)
            ci = pltpu.async_copy(xx_h.at[iv], ri, si)
            cj = pltpu.async_copy(xx_h.at[jv], rj, sj)
            ci.wait()
            cj.wait()
            pltpu.sync_copy(ri, xi_h.at[pl.ds(off, _CH)])
            pltpu.sync_copy(rj, xj_h.at[pl.ds(off, _CH)])
            return 0

        lax.fori_loop(0, cnt, chunk, 0)

    return body(xx, i_idx, j_idx)


def _seg_sum_sc(tab, perm, si, bounds, nslots):
    a = tab.shape[1]
    lp = si.shape[0]
    b8 = bounds.shape[0]
    rows_out = nslots * _RW

    @functools.partial(
        pl.kernel,
        out_type=jax.ShapeDtypeStruct((rows_out, a), jnp.float32),
        mesh=plsc.VectorSubcoreMesh(**_MESH),
        compiler_params=pltpu.CompilerParams(needs_layout_passes=False),
        scratch_types=[
            pltpu.VMEM_SHARED((lp,), jnp.int32),
            pltpu.VMEM_SHARED((lp,), jnp.int32),
            pltpu.VMEM_SHARED((b8,), jnp.int32),
            pltpu.VMEM((_RW + 8, a), jnp.float32),
            pltpu.VMEM((_CH, a), jnp.float32),
            pltpu.VMEM((_CH,), jnp.int32),
            pltpu.SMEM((b8,), jnp.int32),
            pltpu.SMEM((_CH,), jnp.int32),
            pltpu.SemaphoreType.DMA,
        ],
    )
    def body(tab_h, perm_h, si_h, bnd_h, out_h,
             perm_sh, si_sh, bnd_sh, acc, rows, pv, bnd_s, si_s, sem):
        c = lax.axis_index("c")
        t = lax.axis_index("s")
        w = t * 2 + c

        @pl.when(t == 0)
        def _():
            pltpu.sync_copy(perm_h, perm_sh)
            pltpu.sync_copy(si_h, si_sh)
            pltpu.sync_copy(bnd_h, bnd_sh)

        plsc.subcore_barrier()
        pltpu.sync_copy(bnd_sh, bnd_s)

        def slot(iu, _):
            u = w + _NW * iu
            base = u * _RW
            b0 = bnd_s[u]
            b1 = bnd_s[u + 1]

            def zrow(r, _):
                z = jnp.zeros((16,), jnp.float32)
                for cq in range(a // 16):
                    acc[r, pl.ds(16 * cq, 16)] = z
                return 0

            lax.fori_loop(0, _RW + 8, zrow, 0)

            a0 = (b0 // _CH) * _CH
            nch = (b1 - a0 + _CH - 1) // _CH

            def chunk(ic, _):
                off = a0 + ic * _CH
                pltpu.sync_copy(perm_sh.at[pl.ds(off, _CH)], pv)
                pltpu.sync_copy(si_sh.at[pl.ds(off, _CH)], si_s)
                pltpu.async_copy(tab_h.at[pv], rows, sem).wait()

                def line(l, _):
                    li = off + l
                    ok = (li >= b0) & (li < b1)
                    loc = jnp.where(ok, si_s[l] - base, _RW)
                    for cq in range(a // 16):
                        acc[loc, pl.ds(16 * cq, 16)] = (
                            acc[loc, pl.ds(16 * cq, 16)]
                            + rows[l, pl.ds(16 * cq, 16)])
                    return 0

                lax.fori_loop(0, _CH, line, 0)
                return 0

            lax.fori_loop(0, nch, chunk, 0)
            pltpu.sync_copy(acc.at[pl.ds(0, _RW)],
                            out_h.at[pl.ds(base, _RW)])
            return 0

        lax.fori_loop(0, (nslots - w + _NW - 1) // _NW, slot, 0)

    return body(tab, perm, si, bounds)


def _ln(x):
    m = jnp.mean(x, axis=-1, keepdims=True)
    v = jnp.mean((x - m) * (x - m), axis=-1, keepdims=True)
    return (x - m) * lax.rsqrt(v + 1e-5)


def _embed_body(sigma_ref, x_ref, freqs_ref, wemb_ref, bemb_ref, wlx_ref,
                blx_ref, out_ref):
    ang = sigma_ref[...] * freqs_ref[...]
    embin = jnp.concatenate([jnp.cos(ang), jnp.sin(ang)], axis=-1)
    emb = jnp.dot(embin, wemb_ref[...], preferred_element_type=jnp.float32)
    emb = emb + bemb_ref[...]
    xl = jnp.dot(x_ref[...], wlx_ref[...], preferred_element_type=jnp.float32)
    xl = xl + blx_ref[...]
    out_ref[...] = jnp.concatenate([emb, xl], axis=-1)


def _embed(sigma, x, freqs, p_emb, p_lx, rn):
    n, c = sigma.shape[0], freqs.shape[0] * 2
    grid = (n // rn,)
    return pl.pallas_call(
        _embed_body,
        grid=grid,
        in_specs=[
            pl.BlockSpec((rn, 1), lambda i: (i, 0)),
            pl.BlockSpec((rn, 3), lambda i: (i, 0)),
            pl.BlockSpec((1, c // 2), lambda i: (0, 0)),
            pl.BlockSpec((c, c), lambda i: (0, 0)),
            pl.BlockSpec((1, c), lambda i: (0, 0)),
            pl.BlockSpec((3, c), lambda i: (0, 0)),
            pl.BlockSpec((1, c), lambda i: (0, 0)),
        ],
        out_specs=pl.BlockSpec((rn, 2 * c), lambda i: (i, 0)),
        out_shape=jax.ShapeDtypeStruct((n, 2 * c), jnp.float32),
    )(sigma[:, None], x, freqs[None, :], p_emb["W"], p_emb["b"][None, :],
      p_lx["W"], p_lx["b"][None, :])


def _res3_body(a_ref, b_ref, c_ref, w1a, w1b, w1c, b1, w2, b2, out_ref):
    h = (jnp.dot(a_ref[...], w1a[...], preferred_element_type=jnp.float32)
         + jnp.dot(b_ref[...], w1b[...], preferred_element_type=jnp.float32)
         + jnp.dot(c_ref[...], w1c[...], preferred_element_type=jnp.float32)
         + b1[...])
    h = jnp.maximum(h, 0.0)
    y = jnp.dot(h, w2[...], preferred_element_type=jnp.float32) + b2[...] + h
    y = jnp.maximum(y, 0.0)
    out_ref[...] = _ln(y)


def _edge_pre(xi, xj, eij, p1, p2, re):
    e, a = eij.shape
    twoc = xi.shape[1]
    grid = (e // re,)
    w1 = p1["W"]
    return pl.pallas_call(
        _res3_body,
        grid=grid,
        in_specs=[
            pl.BlockSpec((re, twoc), lambda i: (i, 0)),
            pl.BlockSpec((re, twoc), lambda i: (i, 0)),
            pl.BlockSpec((re, a), lambda i: (i, 0)),
            pl.BlockSpec((twoc, a), lambda i: (0, 0)),
            pl.BlockSpec((twoc, a), lambda i: (0, 0)),
            pl.BlockSpec((a, a), lambda i: (0, 0)),
            pl.BlockSpec((1, a), lambda i: (0, 0)),
            pl.BlockSpec((a, a), lambda i: (0, 0)),
            pl.BlockSpec((1, a), lambda i: (0, 0)),
        ],
        out_specs=pl.BlockSpec((re, a), lambda i: (i, 0)),
        out_shape=jax.ShapeDtypeStruct((e, a), jnp.float32),
    )(xi, xj, eij, w1[:twoc], w1[twoc:2 * twoc], w1[2 * twoc:],
      p1["b"][None, :], p2["W"], p2["b"][None, :])


def _qkv_body(e_ref, wq, bq, wk, bk, wv, bv, q_ref, k_ref, v_ref):
    x = e_ref[...]
    sc = 1.0 / math.sqrt(32.0)
    q_ref[...] = (jnp.dot(x, wq[...], preferred_element_type=jnp.float32)
                  + bq[...]) * sc
    k_ref[...] = jnp.dot(x, wk[...], preferred_element_type=jnp.float32) + bk[...]
    v_ref[...] = jnp.dot(x, wv[...], preferred_element_type=jnp.float32) + bv[...]


def _qkv(e_arr, pq, pk, pv, re):
    e, a = e_arr.shape
    grid = (e // re,)
    outs = [jax.ShapeDtypeStruct((e, a), jnp.float32)] * 3
    return pl.pallas_call(
        _qkv_body,
        grid=grid,
        in_specs=[pl.BlockSpec((re, a), lambda i: (i, 0))] + [
            spec for _ in range(3) for spec in
            (pl.BlockSpec((a, a), lambda i: (0, 0)),
             pl.BlockSpec((1, a), lambda i: (0, 0)))],
        out_specs=[pl.BlockSpec((re, a), lambda i: (i, 0))] * 3,
        out_shape=outs,
    )(e_arr, pq["W"], pq["b"][None, :], pk["W"], pk["b"][None, :],
      pv["W"], pv["b"][None, :])


def _post_body(num_ref, den_ref, e_ref, mem_ref, m_ref, wo, bo, w1, b1, w2,
               b2, wu, bu, enew_ref, mem_out_ref):
    den = jnp.dot(den_ref[...], m_ref[...], preferred_element_type=jnp.float32)
    att = num_ref[...] / (den + 1e-9)
    o = jnp.dot(att, wo[...], preferred_element_type=jnp.float32) + bo[...]
    e1 = _ln(e_ref[...] + o)
    h = jnp.dot(e1, w1[...], preferred_element_type=jnp.float32) + b1[...]
    h = jnp.maximum(h, 0.0)
    y = jnp.dot(h, w2[...], preferred_element_type=jnp.float32) + b2[...] + h
    e2 = _ln(jnp.maximum(y, 0.0))
    u = jnp.dot(mem_ref[...], wu[...], preferred_element_type=jnp.float32) + bu[...]
    enew_ref[...] = _ln(e2 + u)
    mem_out_ref[...] = e2


def _post_attn(num, den, e_arr, mem, blk, p_ufo, re):
    e, a = e_arr.shape
    h = 8
    dh = a // h
    m = jnp.repeat(jnp.eye(h, dtype=jnp.float32), dh, axis=1)
    m = jnp.concatenate([m, jnp.zeros((8, a), jnp.float32)], axis=0)
    grid = (e // re,)
    outs = [jax.ShapeDtypeStruct((e, a), jnp.float32)] * 2
    wspec = pl.BlockSpec((a, a), lambda i: (0, 0))
    bspec = pl.BlockSpec((1, a), lambda i: (0, 0))
    return pl.pallas_call(
        _post_body,
        grid=grid,
        in_specs=[
            pl.BlockSpec((re, a), lambda i: (i, 0)),
            pl.BlockSpec((re, 16), lambda i: (i, 0)),
            pl.BlockSpec((re, a), lambda i: (i, 0)),
            pl.BlockSpec((re, a), lambda i: (i, 0)),
            pl.BlockSpec((16, a), lambda i: (0, 0)),
            wspec, bspec, wspec, bspec, wspec, bspec, wspec, bspec,
        ],
        out_specs=[pl.BlockSpec((re, a), lambda i: (i, 0))] * 2,
        out_shape=outs,
    )(num, den, e_arr, mem, m, blk["o"]["W"], blk["o"]["b"][None, :],
      blk["f1"]["W"], blk["f1"]["b"][None, :], blk["f2"]["W"],
      blk["f2"]["b"][None, :], p_ufo["W"], p_ufo["b"][None, :])


def _dqkv_body(xn_ref, wq, bq, wk, bk, wv, bv, x1_ref, q_ref, k_ref, v_ref):
    x1 = _ln(xn_ref[...])
    x1_ref[...] = x1
    sc = 1.0 / math.sqrt(32.0)
    q_ref[...] = (jnp.dot(x1, wq[...], preferred_element_type=jnp.float32)
                  + bq[...]) * sc
    k_ref[...] = jnp.dot(x1, wk[...], preferred_element_type=jnp.float32) + bk[...]
    v_ref[...] = jnp.dot(x1, wv[...], preferred_element_type=jnp.float32) + bv[...]


def _dec_qkv(xn, dec, rn, n):
    a = xn.shape[1]
    grid = (n // rn,)
    outs = [jax.ShapeDtypeStruct((n, a), jnp.float32)] * 4
    return pl.pallas_call(
        _dqkv_body,
        grid=grid,
        in_specs=[pl.BlockSpec((rn, a), lambda i: (i, 0))] + [
            spec for _ in range(3) for spec in
            (pl.BlockSpec((a, a), lambda i: (0, 0)),
             pl.BlockSpec((1, a), lambda i: (0, 0)))],
        out_specs=[pl.BlockSpec((rn, a), lambda i: (i, 0))] * 4,
        out_shape=outs,
    )(xn, dec["q"]["W"], dec["q"]["b"][None, :], dec["k"]["W"],
      dec["k"]["b"][None, :], dec["v"]["W"], dec["v"]["b"][None, :])


def _tail_body(num_ref, den_ref, x1_ref, m_ref, wo, bo, w1, b1, w2, b2,
               out_ref):
    den = jnp.dot(den_ref[...], m_ref[...], preferred_element_type=jnp.float32)
    att = num_ref[...] / (den + 1e-9)
    o = jnp.dot(att, wo[...], preferred_element_type=jnp.float32) + bo[...]
    xo = _ln(o + x1_ref[...])
    h = jnp.dot(xo, w1[...], preferred_element_type=jnp.float32) + b1[...]
    h = jnp.maximum(h, 0.0)
    out_ref[...] = (jnp.dot(h, w2[...], preferred_element_type=jnp.float32)
                    + b2[...] + h)


def _dec_tail(num, den, x1, dec, p_out1, p_out2, rn):
    n, a = x1.shape
    h = 8
    dh = a // h
    m = jnp.repeat(jnp.eye(h, dtype=jnp.float32), dh, axis=1)
    m = jnp.concatenate([m, jnp.zeros((8, a), jnp.float32)], axis=0)
    grid = (n // rn,)
    return pl.pallas_call(
        _tail_body,
        grid=grid,
        in_specs=[
            pl.BlockSpec((rn, a), lambda i: (i, 0)),
            pl.BlockSpec((rn, 16), lambda i: (i, 0)),
            pl.BlockSpec((rn, a), lambda i: (i, 0)),
            pl.BlockSpec((16, a), lambda i: (0, 0)),
            pl.BlockSpec((a, a), lambda i: (0, 0)),
            pl.BlockSpec((1, a), lambda i: (0, 0)),
            pl.BlockSpec((a, 3), lambda i: (0, 0)),
            pl.BlockSpec((1, 3), lambda i: (0, 0)),
            pl.BlockSpec((3, 3), lambda i: (0, 0)),
            pl.BlockSpec((1, 3), lambda i: (0, 0)),
        ],
        out_specs=pl.BlockSpec((rn, 3), lambda i: (i, 0)),
        out_shape=jax.ShapeDtypeStruct((n, 3), jnp.float32),
    )(num, den, x1, m, dec["o"]["W"], dec["o"]["b"][None, :], p_out1["W"],
      p_out1["b"][None, :], p_out2["W"], p_out2["b"][None, :])


def _pad1(v, extra=_CH):
    return jnp.concatenate([v.astype(jnp.int32), jnp.zeros((extra,), jnp.int32)])


def _slot_bounds(si_sorted, nslots):
    steps = jnp.arange(nslots + 1, dtype=jnp.int32) * _RW
    bnd = jnp.searchsorted(si_sorted, steps).astype(jnp.int32)
    b8 = -(-(nslots + 1) // 8) * 8
    return jnp.concatenate(
        [bnd, jnp.broadcast_to(bnd[-1:], (b8 - nslots - 1,))])


def kernel(x, sigma, edge_ij, params, freqs, i_node, j_node, idx_i_edge,
           idx_j_edge):
    n = x.shape[1]
    e = edge_ij.shape[1]
    rn = 1000 if n % 1000 == 0 else n
    re = 3200 if e % 3200 == 0 else e
    nslots_e = -(-e // _RW)
    nslots_n = -(-n // _RW)

    perm2 = jnp.argsort(idx_i_edge)
    si2 = idx_i_edge[perm2]
    sj2 = idx_j_edge[perm2]
    bounds2 = _slot_bounds(si2, nslots_e)
    si2p, sj2p = _pad1(si2), _pad1(sj2)

    perm_n = jnp.argsort(i_node)
    si_n = i_node[perm_n]
    sj_n = j_node[perm_n]
    bounds_n = _slot_bounds(si_n, nslots_n)
    si_np, sj_np, perm_np = _pad1(si_n), _pad1(sj_n), _pad1(perm_n)

    xx = _embed(sigma[0], x[0], freqs, params["emb"], params["lx"], rn)
    xi, xj = _pair_gather_sc(xx, i_node.astype(jnp.int32),
                             j_node.astype(jnp.int32))
    e_arr = _edge_pre(xi, xj, edge_ij[0], params["pre1"], params["pre2"], re)

    mem = e_arr
    for blk in params["blocks"]:
        q, k, v = _qkv(e_arr, blk["q"], blk["k"], blk["v"], re)
        num, den = _seg_attn_sc(q, k, v, si2p, sj2p, bounds2, nslots_e)
        e_arr, mem = _post_attn(num[:e], den.reshape(-1, 16)[:e], e_arr,
                                mem, blk, params["ufo"], re)

    xn = _seg_sum_sc(e_arr, perm_np, si_np, bounds_n, nslots_n)[:n]
    x1, q, k, v = _dec_qkv(xn, params["dec"], rn, n)
    num, den = _seg_attn_sc(q, k, v, si_np, sj_np, bounds_n, nslots_n)
    out = _dec_tail(num[:n], den.reshape(-1, 16)[:n], x1, params["dec"],
                    params["out1"], params["out2"], rn)
    return out[None, :, :]


# speedup vs baseline: 1.0191x; 1.0191x over previous
import functools
import math

import jax
import jax.numpy as jnp
from jax import lax
from jax.experimental import pallas as pl
from jax.experimental.pallas import tpu as pltpu
from jax.experimental.pallas import tpu_sc as plsc

_RW = 192
_CH = 64
_NW = 32
_MESH = dict(core_axis_name="c", subcore_axis_name="s")


def _seg_attn_sc(qt, kt, vt, si, sj, bounds, nslots):
    a = qt.shape[1]
    lp = si.shape[0]
    b8 = bounds.shape[0]
    rows_out = nslots * _RW

    @functools.partial(
        pl.kernel,
        out_type=[jax.ShapeDtypeStruct((rows_out, a), jnp.float32),
                  jax.ShapeDtypeStruct((rows_out * 16,), jnp.float32)],
        mesh=plsc.VectorSubcoreMesh(**_MESH),
        compiler_params=pltpu.CompilerParams(needs_layout_passes=False),
        scratch_types=[
            pltpu.VMEM_SHARED((lp,), jnp.int32),
            pltpu.VMEM_SHARED((b8,), jnp.int32),
            pltpu.VMEM((_RW + 8, a), jnp.float32),
            pltpu.VMEM(((_RW + 8) * 16,), jnp.float32),
            pltpu.VMEM((_CH, a), jnp.float32),
            pltpu.VMEM((_CH, a), jnp.float32),
            pltpu.VMEM((_CH, a), jnp.float32),
            pltpu.VMEM((_CH,), jnp.int32),
            pltpu.VMEM((_CH,), jnp.int32),
            pltpu.SMEM((b8,), jnp.int32),
            pltpu.SMEM((_CH,), jnp.int32),
            pltpu.SemaphoreType.DMA,
            pltpu.SemaphoreType.DMA,
            pltpu.SemaphoreType.DMA,
        ],
    )
    def body(qt_h, kt_h, vt_h, si_h, sj_h, bnd_h, num_h, den_h,
             si_sh, bnd_sh, acc_num, acc_den, qrows, krows, vrows,
             si_v, sj_v, bnd_s, si_s, sem_q, sem_k, sem_v):
        c = lax.axis_index("c")
        t = lax.axis_index("s")
        w = t * 2 + c
        lane = lax.iota(jnp.int32, 16)

        @pl.when(t == 0)
        def _():
            pltpu.sync_copy(si_h, si_sh)
            pltpu.sync_copy(bnd_h, bnd_sh)

        plsc.subcore_barrier()
        pltpu.sync_copy(bnd_sh, bnd_s)

        def slot(iu, _):
            u = w + _NW * iu
            base = u * _RW
            b0 = bnd_s[u]
            b1 = bnd_s[u + 1]

            def zrow(r, _):
                z = jnp.zeros((16,), jnp.float32)
                for cq in range(a // 16):
                    acc_num[r, pl.ds(16 * cq, 16)] = z
                acc_den[pl.ds(16 * r, 16)] = z
                return 0

            lax.fori_loop(0, _RW + 8, zrow, 0)

            a0 = (b0 // _CH) * _CH
            nch = (b1 - a0 + _CH - 1) // _CH

            def chunk(ic, _):
                off = a0 + ic * _CH
                pltpu.sync_copy(si_sh.at[pl.ds(off, _CH)], si_v)
                pltpu.sync_copy(sj_h.at[pl.ds(off, _CH)], sj_v)
                pltpu.sync_copy(si_sh.at[pl.ds(off, _CH)], si_s)
                cq = pltpu.async_copy(qt_h.at[si_v], qrows, sem_q)
                ck = pltpu.async_copy(kt_h.at[sj_v], krows, sem_k)
                cv = pltpu.async_copy(vt_h.at[sj_v], vrows, sem_v)
                cq.wait()
                ck.wait()
                cv.wait()

                def line(p, _):
                    for sub in range(2):
                        l = p * 2 + sub
                        li = off + l
                        ok = (li >= b0) & (li < b1)
                        loc = jnp.where(ok, si_s[l] - base, _RW)
                        acc = jnp.zeros((16,), jnp.float32)
                        for r in range(8):
                            m = (qrows[l, pl.ds(32 * r, 16)]
                                 * krows[l, pl.ds(32 * r, 16)]
                                 + qrows[l, pl.ds(32 * r + 16, 16)]
                                 * krows[l, pl.ds(32 * r + 16, 16)])
                            acc = jnp.where(lane == r, jnp.sum(m), acc)
                        ex = jnp.exp(acc)
                        acc_den[pl.ds(loc * 16, 16)] = (
                            acc_den[pl.ds(loc * 16, 16)] + ex)
                        for r in range(8):
                            er = jnp.sum(jnp.where(lane == r, ex, 0.0))
                            for hh in range(2):
                                co = 32 * r + 16 * hh
                                acc_num[loc, pl.ds(co, 16)] = (
                                    acc_num[loc, pl.ds(co, 16)]
                                    + vrows[l, pl.ds(co, 16)] * er)
                    return 0

                lax.fori_loop(0, _CH // 2, line, 0)
                return 0

            lax.fori_loop(0, nch, chunk, 0)
            pltpu.sync_copy(acc_num.at[pl.ds(0, _RW)],
                            num_h.at[pl.ds(base, _RW)])
            pltpu.sync_copy(acc_den.at[pl.ds(0, _RW * 16)],
                            den_h.at[pl.ds(base * 16, _RW * 16)])
            return 0

        lax.fori_loop(0, (nslots - w + _NW - 1) // _NW, slot, 0)

    return body(qt, kt, vt, si, sj, bounds)


def _pair_gather_sc(xx, i_idx, j_idx):
    e = i_idx.shape[0]
    a = xx.shape[1]
    nch = e // _CH

    @functools.partial(
        pl.kernel,
        out_type=[jax.ShapeDtypeStruct((e, a), jnp.float32),
                  jax.ShapeDtypeStruct((e, a), jnp.float32)],
        mesh=plsc.VectorSubcoreMesh(**_MESH),
        compiler_params=pltpu.CompilerParams(needs_layout_passes=False),
        scratch_types=[
            pltpu.VMEM((_CH,), jnp.int32),
            pltpu.VMEM((_CH,), jnp.int32),
            pltpu.VMEM((_CH, a), jnp.float32),
            pltpu.VMEM((_CH, a), jnp.float32),
            pltpu.SemaphoreType.DMA,
            pltpu.SemaphoreType.DMA,
        ],
    )
    def body(xx_h, ii_h, jj_h, xi_h, xj_h, iv, jv, ri, rj, si, sj):
        c = lax.axis_index("c")
        t = lax.axis_index("s")
        w = t * 2 + c
        cnt = (nch - w + _NW - 1) // _NW

        def chunk(i, _):
            off = (w + i * _NW) * _CH
            pltpu.sync_copy(ii_h.at[pl.ds(off, _CH)], iv)
            pltpu.sync_copy(jj_h.at[pl.ds(off, _CH)], jv)
            ci = pltpu.async_copy(xx_h.at[iv], ri, si)
            cj = pltpu.async_copy(xx_h.at[jv], rj, sj)
            ci.wait()
            cj.wait()
            pltpu.sync_copy(ri, xi_h.at[pl.ds(off, _CH)])
            pltpu.sync_copy(rj, xj_h.at[pl.ds(off, _CH)])
            return 0

        lax.fori_loop(0, cnt, chunk, 0)

    return body(xx, i_idx, j_idx)


def _seg_sum_sc(tab, perm, si, bounds, nslots):
    a = tab.shape[1]
    lp = si.shape[0]
    b8 = bounds.shape[0]
    rows_out = nslots * _RW

    @functools.partial(
        pl.kernel,
        out_type=jax.ShapeDtypeStruct((rows_out, a), jnp.float32),
        mesh=plsc.VectorSubcoreMesh(**_MESH),
        compiler_params=pltpu.CompilerParams(needs_layout_passes=False),
        scratch_types=[
            pltpu.VMEM_SHARED((lp,), jnp.int32),
            pltpu.VMEM_SHARED((lp,), jnp.int32),
            pltpu.VMEM_SHARED((b8,), jnp.int32),
            pltpu.VMEM((_RW + 8, a), jnp.float32),
            pltpu.VMEM((_CH, a), jnp.float32),
            pltpu.VMEM((_CH,), jnp.int32),
            pltpu.SMEM((b8,), jnp.int32),
            pltpu.SMEM((_CH,), jnp.int32),
            pltpu.SemaphoreType.DMA,
        ],
    )
    def body(tab_h, perm_h, si_h, bnd_h, out_h,
             perm_sh, si_sh, bnd_sh, acc, rows, pv, bnd_s, si_s, sem):
        c = lax.axis_index("c")
        t = lax.axis_index("s")
        w = t * 2 + c

        @pl.when(t == 0)
        def _():
            pltpu.sync_copy(perm_h, perm_sh)
            pltpu.sync_copy(si_h, si_sh)
            pltpu.sync_copy(bnd_h, bnd_sh)

        plsc.subcore_barrier()
        pltpu.sync_copy(bnd_sh, bnd_s)

        def slot(iu, _):
            u = w + _NW * iu
            base = u * _RW
            b0 = bnd_s[u]
            b1 = bnd_s[u + 1]

            def zrow(r, _):
                z = jnp.zeros((16,), jnp.float32)
                for cq in range(a // 16):
                    acc[r, pl.ds(16 * cq, 16)] = z
                return 0

            lax.fori_loop(0, _RW + 8, zrow, 0)

            a0 = (b0 // _CH) * _CH
            nch = (b1 - a0 + _CH - 1) // _CH

            def chunk(ic, _):
                off = a0 + ic * _CH
                pltpu.sync_copy(perm_sh.at[pl.ds(off, _CH)], pv)
                pltpu.sync_copy(si_sh.at[pl.ds(off, _CH)], si_s)
                pltpu.async_copy(tab_h.at[pv], rows, sem).wait()

                def line(l, _):
                    li = off + l
                    ok = (li >= b0) & (li < b1)
                    loc = jnp.where(ok, si_s[l] - base, _RW)
                    for cq in range(a // 16):
                        acc[loc, pl.ds(16 * cq, 16)] = (
                            acc[loc, pl.ds(16 * cq, 16)]
                            + rows[l, pl.ds(16 * cq, 16)])
                    return 0

                lax.fori_loop(0, _CH, line, 0)
                return 0

            lax.fori_loop(0, nch, chunk, 0)
            pltpu.sync_copy(acc.at[pl.ds(0, _RW)],
                            out_h.at[pl.ds(base, _RW)])
            return 0

        lax.fori_loop(0, (nslots - w + _NW - 1) // _NW, slot, 0)

    return body(tab, perm, si, bounds)


def _ln(x):
    m = jnp.mean(x, axis=-1, keepdims=True)
    v = jnp.mean((x - m) * (x - m), axis=-1, keepdims=True)
    return (x - m) * lax.rsqrt(v + 1e-5)


def _embed_body(sigma_ref, x_ref, freqs_ref, wemb_ref, bemb_ref, wlx_ref,
                blx_ref, out_ref):
    ang = sigma_ref[...] * freqs_ref[...]
    embin = jnp.concatenate([jnp.cos(ang), jnp.sin(ang)], axis=-1)
    emb = jnp.dot(embin, wemb_ref[...], preferred_element_type=jnp.float32)
    emb = emb + bemb_ref[...]
    xl = jnp.dot(x_ref[...], wlx_ref[...], preferred_element_type=jnp.float32)
    xl = xl + blx_ref[...]
    out_ref[...] = jnp.concatenate([emb, xl], axis=-1)


def _embed(sigma, x, freqs, p_emb, p_lx, rn):
    n, c = sigma.shape[0], freqs.shape[0] * 2
    grid = (n // rn,)
    return pl.pallas_call(
        _embed_body,
        grid=grid,
        in_specs=[
            pl.BlockSpec((rn, 1), lambda i: (i, 0)),
            pl.BlockSpec((rn, 3), lambda i: (i, 0)),
            pl.BlockSpec((1, c // 2), lambda i: (0, 0)),
            pl.BlockSpec((c, c), lambda i: (0, 0)),
            pl.BlockSpec((1, c), lambda i: (0, 0)),
            pl.BlockSpec((3, c), lambda i: (0, 0)),
            pl.BlockSpec((1, c), lambda i: (0, 0)),
        ],
        out_specs=pl.BlockSpec((rn, 2 * c), lambda i: (i, 0)),
        out_shape=jax.ShapeDtypeStruct((n, 2 * c), jnp.float32),
    )(sigma[:, None], x, freqs[None, :], p_emb["W"], p_emb["b"][None, :],
      p_lx["W"], p_lx["b"][None, :])


def _res3_body(a_ref, b_ref, c_ref, w1a, w1b, w1c, b1, w2, b2, out_ref):
    h = (jnp.dot(a_ref[...], w1a[...], preferred_element_type=jnp.float32)
         + jnp.dot(b_ref[...], w1b[...], preferred_element_type=jnp.float32)
         + jnp.dot(c_ref[...], w1c[...], preferred_element_type=jnp.float32)
         + b1[...])
    h = jnp.maximum(h, 0.0)
    y = jnp.dot(h, w2[...], preferred_element_type=jnp.float32) + b2[...] + h
    y = jnp.maximum(y, 0.0)
    out_ref[...] = _ln(y)


def _edge_pre(xi, xj, eij, p1, p2, re):
    e, a = eij.shape
    twoc = xi.shape[1]
    grid = (e // re,)
    w1 = p1["W"]
    return pl.pallas_call(
        _res3_body,
        grid=grid,
        in_specs=[
            pl.BlockSpec((re, twoc), lambda i: (i, 0)),
            pl.BlockSpec((re, twoc), lambda i: (i, 0)),
            pl.BlockSpec((re, a), lambda i: (i, 0)),
            pl.BlockSpec((twoc, a), lambda i: (0, 0)),
            pl.BlockSpec((twoc, a), lambda i: (0, 0)),
            pl.BlockSpec((a, a), lambda i: (0, 0)),
            pl.BlockSpec((1, a), lambda i: (0, 0)),
            pl.BlockSpec((a, a), lambda i: (0, 0)),
            pl.BlockSpec((1, a), lambda i: (0, 0)),
        ],
        out_specs=pl.BlockSpec((re, a), lambda i: (i, 0)),
        out_shape=jax.ShapeDtypeStruct((e, a), jnp.float32),
    )(xi, xj, eij, w1[:twoc], w1[twoc:2 * twoc], w1[2 * twoc:],
      p1["b"][None, :], p2["W"], p2["b"][None, :])


def _qkv_body(e_ref, wq, bq, wk, bk, wv, bv, q_ref, k_ref, v_ref):
    x = e_ref[...]
    sc = 1.0 / math.sqrt(32.0)
    q_ref[...] = (jnp.dot(x, wq[...], preferred_element_type=jnp.float32)
                  + bq[...]) * sc
    k_ref[...] = jnp.dot(x, wk[...], preferred_element_type=jnp.float32) + bk[...]
    v_ref[...] = jnp.dot(x, wv[...], preferred_element_type=jnp.float32) + bv[...]


def _qkv(e_arr, pq, pk, pv, re):
    e, a = e_arr.shape
    grid = (e // re,)
    outs = [jax.ShapeDtypeStruct((e, a), jnp.float32)] * 3
    return pl.pallas_call(
        _qkv_body,
        grid=grid,
        in_specs=[pl.BlockSpec((re, a), lambda i: (i, 0))] + [
            spec for _ in range(3) for spec in
            (pl.BlockSpec((a, a), lambda i: (0, 0)),
             pl.BlockSpec((1, a), lambda i: (0, 0)))],
        out_specs=[pl.BlockSpec((re, a), lambda i: (i, 0))] * 3,
        out_shape=outs,
    )(e_arr, pq["W"], pq["b"][None, :], pk["W"], pk["b"][None, :],
      pv["W"], pv["b"][None, :])


def _post_body(num_ref, den_ref, e_ref, mem_ref, m_ref, wo, bo, w1, b1, w2,
               b2, wu, bu, enew_ref, mem_out_ref):
    den = jnp.dot(den_ref[...], m_ref[...], preferred_element_type=jnp.float32)
    att = num_ref[...] / (den + 1e-9)
    o = jnp.dot(att, wo[...], preferred_element_type=jnp.float32) + bo[...]
    e1 = _ln(e_ref[...] + o)
    h = jnp.dot(e1, w1[...], preferred_element_type=jnp.float32) + b1[...]
    h = jnp.maximum(h, 0.0)
    y = jnp.dot(h, w2[...], preferred_element_type=jnp.float32) + b2[...] + h
    e2 = _ln(jnp.maximum(y, 0.0))
    u = jnp.dot(mem_ref[...], wu[...], preferred_element_type=jnp.float32) + bu[...]
    enew_ref[...] = _ln(e2 + u)
    mem_out_ref[...] = e2


def _post_attn(num, den, e_arr, mem, blk, p_ufo, re):
    e, a = e_arr.shape
    h = 8
    dh = a // h
    m = jnp.repeat(jnp.eye(h, dtype=jnp.float32), dh, axis=1)
    m = jnp.concatenate([m, jnp.zeros((8, a), jnp.float32)], axis=0)
    grid = (e // re,)
    outs = [jax.ShapeDtypeStruct((e, a), jnp.float32)] * 2
    wspec = pl.BlockSpec((a, a), lambda i: (0, 0))
    bspec = pl.BlockSpec((1, a), lambda i: (0, 0))
    return pl.pallas_call(
        _post_body,
        grid=grid,
        in_specs=[
            pl.BlockSpec((re, a), lambda i: (i, 0)),
            pl.BlockSpec((re, 16), lambda i: (i, 0)),
            pl.BlockSpec((re, a), lambda i: (i, 0)),
            pl.BlockSpec((re, a), lambda i: (i, 0)),
            pl.BlockSpec((16, a), lambda i: (0, 0)),
            wspec, bspec, wspec, bspec, wspec, bspec, wspec, bspec,
        ],
        out_specs=[pl.BlockSpec((re, a), lambda i: (i, 0))] * 2,
        out_shape=outs,
    )(num, den, e_arr, mem, m, blk["o"]["W"], blk["o"]["b"][None, :],
      blk["f1"]["W"], blk["f1"]["b"][None, :], blk["f2"]["W"],
      blk["f2"]["b"][None, :], p_ufo["W"], p_ufo["b"][None, :])


def _dqkv_body(xn_ref, wq, bq, wk, bk, wv, bv, x1_ref, q_ref, k_ref, v_ref):
    x1 = _ln(xn_ref[...])
    x1_ref[...] = x1
    sc = 1.0 / math.sqrt(32.0)
    q_ref[...] = (jnp.dot(x1, wq[...], preferred_element_type=jnp.float32)
                  + bq[...]) * sc
    k_ref[...] = jnp.dot(x1, wk[...], preferred_element_type=jnp.float32) + bk[...]
    v_ref[...] = jnp.dot(x1, wv[...], preferred_element_type=jnp.float32) + bv[...]


def _dec_qkv(xn, dec, rn, n):
    a = xn.shape[1]
    grid = (n // rn,)
    outs = [jax.ShapeDtypeStruct((n, a), jnp.float32)] * 4
    return pl.pallas_call(
        _dqkv_body,
        grid=grid,
        in_specs=[pl.BlockSpec((rn, a), lambda i: (i, 0))] + [
            spec for _ in range(3) for spec in
            (pl.BlockSpec((a, a), lambda i: (0, 0)),
             pl.BlockSpec((1, a), lambda i: (0, 0)))],
        out_specs=[pl.BlockSpec((rn, a), lambda i: (i, 0))] * 4,
        out_shape=outs,
    )(xn, dec["q"]["W"], dec["q"]["b"][None, :], dec["k"]["W"],
      dec["k"]["b"][None, :], dec["v"]["W"], dec["v"]["b"][None, :])


def _tail_body(num_ref, den_ref, x1_ref, m_ref, wo, bo, w1, b1, w2, b2,
               out_ref):
    den = jnp.dot(den_ref[...], m_ref[...], preferred_element_type=jnp.float32)
    att = num_ref[...] / (den + 1e-9)
    o = jnp.dot(att, wo[...], preferred_element_type=jnp.float32) + bo[...]
    xo = _ln(o + x1_ref[...])
    h = jnp.dot(xo, w1[...], preferred_element_type=jnp.float32) + b1[...]
    h = jnp.maximum(h, 0.0)
    out_ref[...] = (jnp.dot(h, w2[...], preferred_element_type=jnp.float32)
                    + b2[...] + h)


def _dec_tail(num, den, x1, dec, p_out1, p_out2, rn):
    n, a = x1.shape
    h = 8
    dh = a // h
    m = jnp.repeat(jnp.eye(h, dtype=jnp.float32), dh, axis=1)
    m = jnp.concatenate([m, jnp.zeros((8, a), jnp.float32)], axis=0)
    grid = (n // rn,)
    return pl.pallas_call(
        _tail_body,
        grid=grid,
        in_specs=[
            pl.BlockSpec((rn, a), lambda i: (i, 0)),
            pl.BlockSpec((rn, 16), lambda i: (i, 0)),
            pl.BlockSpec((rn, a), lambda i: (i, 0)),
            pl.BlockSpec((16, a), lambda i: (0, 0)),
            pl.BlockSpec((a, a), lambda i: (0, 0)),
            pl.BlockSpec((1, a), lambda i: (0, 0)),
            pl.BlockSpec((a, 3), lambda i: (0, 0)),
            pl.BlockSpec((1, 3), lambda i: (0, 0)),
            pl.BlockSpec((3, 3), lambda i: (0, 0)),
            pl.BlockSpec((1, 3), lambda i: (0, 0)),
        ],
        out_specs=pl.BlockSpec((rn, 3), lambda i: (i, 0)),
        out_shape=jax.ShapeDtypeStruct((n, 3), jnp.float32),
    )(num, den, x1, m, dec["o"]["W"], dec["o"]["b"][None, :], p_out1["W"],
      p_out1["b"][None, :], p_out2["W"], p_out2["b"][None, :])


def _pad1(v, extra=_CH):
    return jnp.concatenate([v.astype(jnp.int32), jnp.zeros((extra,), jnp.int32)])


def _slot_bounds(si_sorted, nslots):
    steps = jnp.arange(nslots + 1, dtype=jnp.int32) * _RW
    bnd = jnp.searchsorted(si_sorted, steps).astype(jnp.int32)
    b8 = -(-(nslots + 1) // 8) * 8
    return jnp.concatenate(
        [bnd, jnp.broadcast_to(bnd[-1:], (b8 - nslots - 1,))])


def kernel(x, sigma, edge_ij, params, freqs, i_node, j_node, idx_i_edge,
           idx_j_edge):
    n = x.shape[1]
    e = edge_ij.shape[1]
    rn = 1000 if n % 1000 == 0 else n
    re = 3200 if e % 3200 == 0 else e
    nslots_e = -(-e // _RW)
    nslots_n = -(-n // _RW)

    perm2 = jnp.argsort(idx_i_edge)
    si2 = idx_i_edge[perm2]
    sj2 = idx_j_edge[perm2]
    bounds2 = _slot_bounds(si2, nslots_e)
    si2p, sj2p = _pad1(si2), _pad1(sj2)

    perm_n = jnp.argsort(i_node)
    si_n = i_node[perm_n]
    sj_n = j_node[perm_n]
    bounds_n = _slot_bounds(si_n, nslots_n)
    si_np, sj_np, perm_np = _pad1(si_n), _pad1(sj_n), _pad1(perm_n)

    xx = _embed(sigma[0], x[0], freqs, params["emb"], params["lx"], rn)
    xi, xj = _pair_gather_sc(xx, i_node.astype(jnp.int32),
                             j_node.astype(jnp.int32))
    e_arr = _edge_pre(xi, xj, edge_ij[0], params["pre1"], params["pre2"], re)

    mem = e_arr
    for blk in params["blocks"]:
        q, k, v = _qkv(e_arr, blk["q"], blk["k"], blk["v"], re)
        num, den = _seg_attn_sc(q, k, v, si2p, sj2p, bounds2, nslots_e)
        e_arr, mem = _post_attn(num[:e], den.reshape(-1, 16)[:e], e_arr,
                                mem, blk, params["ufo"], re)

    xn = _seg_sum_sc(e_arr, perm_np, si_np, bounds_n, nslots_n)[:n]
    x1, q, k, v = _dec_qkv(xn, params["dec"], rn, n)
    num, den = _seg_attn_sc(q, k, v, si_np, sj_np, bounds_n, nslots_n)
    out = _dec_tail(num[:n], den.reshape(-1, 16)[:n], x1, params["dec"],
                    params["out1"], params["out2"], rn)
    return out[None, :, :]
